# baseline (speedup 1.0000x reference)
import jax
import jax.numpy as jnp
from jax import lax
from jax.experimental import pallas as pl
from jax.experimental.pallas import tpu as pltpu
from jax.experimental.pallas import tpu_sc as plsc

_NUM_CORES = 2
_NUM_SUBCORES = 16
_NUM_WORKERS = _NUM_CORES * _NUM_SUBCORES
_REP_SH = 32


def _make_sc_body(block, per_worker, rep):
    n_dma = per_worker // rep
    copies_per_tile = rep // _NUM_SUBCORES

    def body(table_hbm, out_hbm, tile_buf, shared, sem):
        cid = lax.axis_index("c")
        sid = lax.axis_index("s")
        wid = sid * _NUM_CORES + cid
        base = wid * (per_worker * block)
        pltpu.sync_copy(table_hbm.at[pl.ds(0, block)], tile_buf)
        for t in range(copies_per_tile):
            idx = sid * copies_per_tile + t
            pltpu.sync_copy(tile_buf, shared.at[pl.ds(idx * block, block)])
        plsc.subcore_barrier()
        copies = []
        for j in range(n_dma):
            copies.append(
                pltpu.async_copy(
                    shared.at[pl.ds(0, rep * block)],
                    out_hbm.at[pl.ds(base + j * rep * block, rep * block)],
                    sem,
                )
            )
        for c in copies:
            c.wait()

    return body


def kernel(x, pos_table):
    N, S = x.shape
    D = pos_table.shape[1]
    block = S * D
    per_worker = N // _NUM_WORKERS
    assert per_worker * _NUM_WORKERS == N and block % 8 == 0
    rep = _REP_SH
    while per_worker % rep or rep % _NUM_SUBCORES:
        rep //= 2
    assert rep >= _NUM_SUBCORES

    mesh = plsc.VectorSubcoreMesh(core_axis_name="c", subcore_axis_name="s")
    k = pl.kernel(
        _make_sc_body(block, per_worker, rep),
        out_type=jax.ShapeDtypeStruct((N * block,), jnp.float32),
        mesh=mesh,
        scratch_types=[
            pltpu.VMEM((block,), jnp.float32),
            pltpu.VMEM_SHARED((rep * block,), jnp.float32),
            pltpu.SemaphoreType.DMA,
        ],
    )
    flat = k(pos_table.reshape(-1))
    return flat.reshape(N, S, D)


# speedup vs baseline: 1.3742x; 1.3742x over previous
import jax
import jax.numpy as jnp
from jax import lax
from jax.experimental import pallas as pl
from jax.experimental.pallas import tpu as pltpu
from jax.experimental.pallas import tpu_sc as plsc

_NUM_CORES = 2
_NUM_SUBCORES = 16
_NUM_WORKERS = _NUM_CORES * _NUM_SUBCORES
_REP = 4


def _make_sc_body(S, per_worker, rep):
    n_dma = per_worker // rep

    def body(table_hbm, out_hbm, buf, sem):
        wid = lax.axis_index("s") * _NUM_CORES + lax.axis_index("c")
        base = wid * per_worker
        for i in range(rep):
            pltpu.sync_copy(table_hbm.at[pl.ds(0, S)], buf.at[i])
        copies = []
        for j in range(n_dma):
            copies.append(
                pltpu.async_copy(
                    buf, out_hbm.at[pl.ds(base + j * rep, rep)], sem
                )
            )
        for c in copies:
            c.wait()

    return body


def kernel(x, pos_table):
    N, S = x.shape
    D = pos_table.shape[1]
    per_worker = N // _NUM_WORKERS
    assert per_worker * _NUM_WORKERS == N
    rep = _REP
    while per_worker % rep:
        rep //= 2

    mesh = plsc.VectorSubcoreMesh(core_axis_name="c", subcore_axis_name="s")
    k = pl.kernel(
        _make_sc_body(S, per_worker, rep),
        out_type=jax.ShapeDtypeStruct((N, S, D), jnp.float32),
        mesh=mesh,
        scratch_types=[
            pltpu.VMEM((rep, S, D), jnp.float32),
            pltpu.SemaphoreType.DMA,
        ],
    )
    return k(pos_table)


# speedup vs baseline: 1.3743x; 1.0001x over previous
import jax
import jax.numpy as jnp
from jax import lax
from jax.experimental import pallas as pl
from jax.experimental.pallas import tpu as pltpu
from jax.experimental.pallas import tpu_sc as plsc

_NUM_CORES = 2
_NUM_SUBCORES = 16
_NUM_WORKERS = _NUM_CORES * _NUM_SUBCORES
_REP = 4


def _make_sc_body(S, per_worker, rep):
    n_dma = per_worker // rep

    def body(table_hbm, out_hbm, buf, sem):
        wid = lax.axis_index("s") * _NUM_CORES + lax.axis_index("c")
        base = wid * per_worker
        for i in range(rep):
            pltpu.sync_copy(table_hbm.at[pl.ds(0, S)], buf.at[i])
        copies = []
        for j in range(n_dma):
            copies.append(
                pltpu.async_copy(
                    buf, out_hbm.at[pl.ds(base + j * rep, rep)], sem
                )
            )
        for c in copies:
            c.wait()

    return body


def kernel(x, pos_table):
    N, S = x.shape
    D = pos_table.shape[1]
    per_worker = N // _NUM_WORKERS
    assert per_worker * _NUM_WORKERS == N
    rep = _REP
    while per_worker % rep:
        rep //= 2

    mesh = plsc.VectorSubcoreMesh(core_axis_name="c", subcore_axis_name="s")
    k = pl.kernel(
        _make_sc_body(S, per_worker, rep),
        out_type=jax.ShapeDtypeStruct((N, S, D), jnp.float32),
        mesh=mesh,
        scratch_types=[
            pltpu.VMEM((rep, S, D), jnp.float32),
            pltpu.SemaphoreType.DMA,
        ],
        compiler_params=pltpu.CompilerParams(use_tc_tiling_on_sc=True),
    )
    return k(pos_table)
